# baseline (speedup 1.0000x reference)
import functools

import jax
import jax.numpy as jnp
from jax import lax
from jax.experimental import pallas as pl
from jax.experimental.pallas import tpu as pltpu
from jax.experimental.pallas import tpu_sc as plsc

VOCAB = 100000
EMBED_DIM = 16
BATCH = 16384

_NC = 2
_NS = 16
_NW = _NC * _NS
_LANES = 16

_CHUNK = 128
_B_PER_W = BATCH // _NW
_N_CHUNKS = _B_PER_W // _CHUNK


def _lookup_kernel(idx_hbm, table_hbm, out_hbm, idx_v, rows_v, sem):
    wid = lax.axis_index("s") * _NC + lax.axis_index("c")
    base = wid * _B_PER_W

    pltpu.sync_copy(idx_hbm.at[pl.ds(base, _B_PER_W)], idx_v)

    def remap(i, carry):
        sl = pl.ds(i * _LANES, _LANES)
        v = idx_v[sl]
        idx_v[sl] = jnp.where((v >= 0) & (v < VOCAB), v + 1, 0)
        return carry

    lax.fori_loop(0, _B_PER_W // _LANES, remap, 0)

    def fire(j, carry):
        sl = pl.ds(j * _CHUNK, _CHUNK)
        pltpu.async_copy(table_hbm.at[idx_v.at[sl]], rows_v.at[sl], sem)
        return carry

    lax.fori_loop(0, _N_CHUNKS, fire, 0)

    pltpu.make_async_copy(table_hbm.at[pl.ds(0, _B_PER_W)], rows_v, sem).wait()

    pltpu.sync_copy(rows_v, out_hbm.at[pl.ds(base, _B_PER_W)])


def kernel(user, table):
    mesh = plsc.VectorSubcoreMesh(core_axis_name="c", subcore_axis_name="s")
    run = functools.partial(
        pl.kernel,
        mesh=mesh,
        compiler_params=pltpu.CompilerParams(
            use_tc_tiling_on_sc=False,
            disable_bounds_checks=True,
            disable_semaphore_checks=True,
        ),
        out_type=jax.ShapeDtypeStruct((BATCH, EMBED_DIM), jnp.float32),
        scratch_types=[
            pltpu.VMEM((_B_PER_W,), jnp.int32),
            pltpu.VMEM((_B_PER_W, EMBED_DIM), jnp.float32),
            pltpu.SemaphoreType.DMA,
        ],
    )(_lookup_kernel)
    return run(user.astype(jnp.int32), table)


# speedup vs baseline: 1.7579x; 1.7579x over previous
import functools

import jax
import jax.numpy as jnp
from jax import lax
from jax.experimental import pallas as pl
from jax.experimental.pallas import tpu as pltpu
from jax.experimental.pallas import tpu_sc as plsc

VOCAB = 100000
EMBED_DIM = 16
BATCH = 16384

_NC = 2
_NS = 16
_NW = _NC * _NS
_LANES = 16

_CHUNK = 128
_B_PER_W = BATCH // _NW
_N_CHUNKS = _B_PER_W // _CHUNK
_STRIDE = VOCAB + 1


def _lookup_kernel(idx_hbm, tab_hbm, out_hbm, idx_v, eidx_v, dst_v, sem):
    wid = lax.axis_index("s") * _NC + lax.axis_index("c")
    base = wid * _B_PER_W

    pltpu.sync_copy(idx_hbm.at[pl.ds(base, _B_PER_W)], idx_v)

    def remap(i, carry):
        sl = pl.ds(i * _LANES, _LANES)
        v = idx_v[sl]
        idx_v[sl] = jnp.where((v >= 0) & (v < VOCAB), v + 1, 0)
        return carry

    lax.fori_loop(0, _B_PER_W // _LANES, remap, 0)

    def do_chunk(c, carry):
        def gen(i, carry2):
            d = i // (_CHUNK // _LANES)
            g = i % (_CHUNK // _LANES)
            rv = idx_v[pl.ds(c * _CHUNK + g * _LANES, _LANES)]
            eidx_v[d, pl.ds(g * _LANES, _LANES)] = rv + d * _STRIDE
            return carry2

        lax.fori_loop(0, EMBED_DIM * (_CHUNK // _LANES), gen, 0)

        def fire(d, carry2):
            pltpu.async_copy(tab_hbm.at[eidx_v.at[d]], dst_v.at[d], sem)
            return carry2

        lax.fori_loop(0, EMBED_DIM, fire, 0)

        pltpu.make_async_copy(out_hbm.at[:, pl.ds(0, _CHUNK)], dst_v,
                              sem).wait()

        pltpu.sync_copy(dst_v, out_hbm.at[:, pl.ds(base + c * _CHUNK,
                                                   _CHUNK)])
        return carry

    lax.fori_loop(0, _N_CHUNKS, do_chunk, 0)


def kernel(user, table):
    mesh = plsc.VectorSubcoreMesh(core_axis_name="c", subcore_axis_name="s")
    run = functools.partial(
        pl.kernel,
        mesh=mesh,
        compiler_params=pltpu.CompilerParams(
            use_tc_tiling_on_sc=False,
            disable_bounds_checks=True,
            disable_semaphore_checks=True,
        ),
        out_type=jax.ShapeDtypeStruct((EMBED_DIM, BATCH), jnp.float32),
        scratch_types=[
            pltpu.VMEM((_B_PER_W,), jnp.int32),
            pltpu.VMEM((EMBED_DIM, _CHUNK), jnp.int32),
            pltpu.VMEM((EMBED_DIM, _CHUNK), jnp.float32),
            pltpu.SemaphoreType.DMA,
        ],
    )(_lookup_kernel)
    tab_flat = table.T.reshape(-1)
    out_t = run(user.astype(jnp.int32), tab_flat)
    return out_t.T


# speedup vs baseline: 1.9437x; 1.1057x over previous
import functools

import jax
import jax.numpy as jnp
from jax import lax
from jax.experimental import pallas as pl
from jax.experimental.pallas import tpu as pltpu
from jax.experimental.pallas import tpu_sc as plsc

VOCAB = 100000
EMBED_DIM = 16
BATCH = 16384

_NC = 2
_NS = 16
_NW = _NC * _NS
_LANES = 16

_CHUNK = 128
_B_PER_W = BATCH // _NW
_N_CHUNKS = _B_PER_W // _CHUNK
_STRIDE = VOCAB + 1


def _lookup_kernel(idx_hbm, tab_hbm, out_hbm, idx_v, eidx_v, dst_v, sem):
    wid = lax.axis_index("s") * _NC + lax.axis_index("c")
    base = wid * _B_PER_W

    pltpu.sync_copy(idx_hbm.at[pl.ds(base, _B_PER_W)], idx_v)

    def remap(i, carry):
        sl = pl.ds(i * _LANES, _LANES)
        v = idx_v[sl]
        idx_v[sl] = jnp.where((v >= 0) & (v < VOCAB), v + 1, 0)
        return carry

    lax.fori_loop(0, _B_PER_W // _LANES, remap, 0)

    def do_chunk(c, carry):
        def gen(i, carry2):
            d = i // (_CHUNK // _LANES)
            g = i % (_CHUNK // _LANES)
            rv = idx_v[pl.ds(c * _CHUNK + g * _LANES, _LANES)]
            eidx_v[d, pl.ds(c * _CHUNK + g * _LANES, _LANES)] = (
                rv + d * _STRIDE)
            return carry2

        lax.fori_loop(0, EMBED_DIM * (_CHUNK // _LANES), gen, 0)

        def fire(d, carry2):
            sl = pl.ds(c * _CHUNK, _CHUNK)
            pltpu.async_copy(tab_hbm.at[eidx_v.at[d, sl]], dst_v.at[d, sl],
                             sem)
            return carry2

        lax.fori_loop(0, EMBED_DIM, fire, 0)
        return carry

    lax.fori_loop(0, _N_CHUNKS, do_chunk, 0)

    pltpu.make_async_copy(out_hbm.at[:, pl.ds(0, _B_PER_W)], dst_v,
                          sem).wait()
    pltpu.sync_copy(dst_v, out_hbm.at[:, pl.ds(base, _B_PER_W)])


def kernel(user, table):
    mesh = plsc.VectorSubcoreMesh(core_axis_name="c", subcore_axis_name="s")
    run = functools.partial(
        pl.kernel,
        mesh=mesh,
        compiler_params=pltpu.CompilerParams(
            use_tc_tiling_on_sc=False,
            disable_bounds_checks=True,
            disable_semaphore_checks=True,
        ),
        out_type=jax.ShapeDtypeStruct((EMBED_DIM, BATCH), jnp.float32),
        scratch_types=[
            pltpu.VMEM((_B_PER_W,), jnp.int32),
            pltpu.VMEM((EMBED_DIM, _B_PER_W), jnp.int32),
            pltpu.VMEM((EMBED_DIM, _B_PER_W), jnp.float32),
            pltpu.SemaphoreType.DMA,
        ],
    )(_lookup_kernel)
    tab_flat = table.T.reshape(-1)
    out_t = run(user.astype(jnp.int32), tab_flat)
    return out_t.T
